# baseline (speedup 1.0000x reference)
import functools

import jax
import jax.numpy as jnp
from jax import lax
from jax.experimental import pallas as pl
from jax.experimental.pallas import tpu as pltpu
from jax.experimental.pallas import tpu_sc as plsc

_SC_DELAYS = ((3, 8), (4, 16), (5, 32))
_TC_DELAYS = ((0, 1), (1, 2), (2, 4))
_K = 6
_COL = 256
_CH = 32
_NCH = _COL // _CH
_TC_BLOCK = 256


def _sc_part(embeddings):
    B, S, D = embeddings.shape

    info = plsc.get_sparse_core_info()
    nw = info.num_cores * info.num_subcores
    ncol = S // _COL
    assert nw == B * ncol
    mesh = plsc.VectorSubcoreMesh(core_axis_name="c", subcore_axis_name="s")

    @functools.partial(
        pl.kernel,
        out_type=jax.ShapeDtypeStruct((B, S, _K * D), jnp.float32),
        mesh=mesh,
        scratch_types=[
            pltpu.VMEM((3 * _CH, D), jnp.float32),
            pltpu.SemaphoreType.DMA,
            pltpu.SemaphoreType.DMA,
            pltpu.SemaphoreType.DMA,
        ],
    )
    def run(emb_hbm, out_hbm, ring, sem0, sem1, sem2):
        cid = lax.axis_index("c")
        sid = lax.axis_index("s")
        wid = sid * info.num_cores + cid
        b = wid // ncol
        c = wid % ncol
        t0 = c * _COL
        sems = (sem0, sem1, sem2)

        def win(i):
            return ring.at[pl.ds(((i + 1) % 3) * _CH, _CH)]

        def chunk_copies(i, do):
            sem = sems[i % 3]
            r0 = t0 + i * _CH
            adj = i % 3 < 2
            for k, d in _SC_DELAYS:
                ksl = pl.ds(k * D, D)
                if i == 0:
                    @pl.when(c > 0)
                    def _body(k=k, d=d, ksl=ksl):
                        if d == _CH:
                            do(win(-1), out_hbm.at[b, pl.ds(r0, _CH), ksl], sem)
                        else:
                            do(ring.at[pl.ds(_CH - d, _CH)],
                               out_hbm.at[b, pl.ds(r0, _CH), ksl], sem)

                    @pl.when(c == 0)
                    def _head(k=k, d=d, ksl=ksl):
                        if d == _CH:
                            do(win(0), out_hbm.at[b, pl.ds(0, _CH), ksl], sem)
                        else:
                            do(win(0).at[pl.ds(0, d)],
                               out_hbm.at[b, pl.ds(0, d), ksl], sem)
                            do(win(0).at[pl.ds(0, _CH - d)],
                               out_hbm.at[b, pl.ds(d, _CH - d), ksl], sem)
                else:
                    if d == _CH:
                        do(win(i - 1), out_hbm.at[b, pl.ds(r0, _CH), ksl], sem)
                    elif adj:
                        do(ring.at[pl.ds((i % 3) * _CH + _CH - d, _CH)],
                           out_hbm.at[b, pl.ds(r0, _CH), ksl], sem)
                    else:
                        do(win(i - 1).at[pl.ds(_CH - d, d)],
                           out_hbm.at[b, pl.ds(r0, d), ksl], sem)
                        do(win(i).at[pl.ds(0, _CH - d)],
                           out_hbm.at[b, pl.ds(r0 + d, _CH - d), ksl], sem)

        def issue(src, dst, sem):
            pltpu.async_copy(src, dst, sem)

        def drain(src, dst, sem):
            pltpu.make_async_copy(src, dst, sem).wait()

        @pl.when(c > 0)
        def _halo():
            pltpu.sync_copy(emb_hbm.at[b, pl.ds(t0 - _CH, _CH), :], win(-1))

        pltpu.sync_copy(emb_hbm.at[b, pl.ds(t0, _CH), :], win(0))
        chunk_copies(0, issue)
        for i in range(1, _NCH):
            if i >= 2:
                chunk_copies(i - 2, drain)
            pltpu.sync_copy(
                emb_hbm.at[b, pl.ds(t0 + i * _CH, _CH), :], win(i))
            chunk_copies(i, issue)

        chunk_copies(_NCH - 2, drain)
        chunk_copies(_NCH - 1, drain)

    return run(embeddings)


def _tc_kernel(emb_ref, halo_ref, out_sc_ref, out_ref):
    del out_sc_ref
    i = pl.program_id(1)
    T = _TC_BLOCK
    cur = emb_ref[0]
    halo = halo_ref[0]
    D = cur.shape[1]
    for k, d in _TC_DELAYS:
        out_ref[0, d:, k * D:(k + 1) * D] = cur[:T - d]

        @pl.when(i == 0)
        def _head(k=k, d=d):
            out_ref[0, :d, k * D:(k + 1) * D] = cur[:d]

        @pl.when(i != 0)
        def _carry(k=k, d=d):
            out_ref[0, :d, k * D:(k + 1) * D] = halo[8 - d:]


def kernel(embeddings):
    B, S, D = embeddings.shape
    T = _TC_BLOCK
    out_sc = _sc_part(embeddings)

    return pl.pallas_call(
        _tc_kernel,
        grid=(B, S // T),
        in_specs=[
            pl.BlockSpec((1, T, D), lambda b, i: (b, i, 0)),
            pl.BlockSpec((1, 8, D),
                         lambda b, i: (b, jnp.maximum(i * (T // 8) - 1, 0), 0)),
            pl.BlockSpec(memory_space=pl.ANY),
        ],
        out_specs=pl.BlockSpec((1, T, 3 * D), lambda b, i: (b, i, 0)),
        out_shape=jax.ShapeDtypeStruct((B, S, _K * D), jnp.float32),
        input_output_aliases={2: 0},
    )(embeddings, embeddings, out_sc)


# speedup vs baseline: 1.0076x; 1.0076x over previous
import functools

import jax
import jax.numpy as jnp
from jax import lax
from jax.experimental import pallas as pl
from jax.experimental.pallas import tpu as pltpu
from jax.experimental.pallas import tpu_sc as plsc

_SC_DELAYS = ((3, 8), (4, 16), (5, 32))
_TC_DELAYS = ((0, 1), (1, 2), (2, 4))
_K = 6
_COL = 256
_CH = 32
_NCH = _COL // _CH
_TC_BLOCK = 256


def _sc_part(embeddings):
    B, S, D = embeddings.shape

    info = plsc.get_sparse_core_info()
    nw = info.num_cores * info.num_subcores
    ncol = S // _COL
    assert nw == B * ncol
    mesh = plsc.VectorSubcoreMesh(core_axis_name="c", subcore_axis_name="s")

    @functools.partial(
        pl.kernel,
        out_type=jax.ShapeDtypeStruct((B, S, _K * D), jnp.float32),
        mesh=mesh,
        scratch_types=[
            pltpu.VMEM((3 * _CH, D), jnp.float32),
            pltpu.SemaphoreType.DMA,
            pltpu.SemaphoreType.DMA,
            pltpu.SemaphoreType.DMA,
        ],
    )
    def run(emb_hbm, out_hbm, ring, sem0, sem1, sem2):
        cid = lax.axis_index("c")
        sid = lax.axis_index("s")
        wid = sid * info.num_cores + cid
        b = wid // ncol
        c = wid % ncol
        t0 = c * _COL
        sems = (sem0, sem1, sem2)

        def win(i):
            return ring.at[pl.ds(((i + 1) % 3) * _CH, _CH)]

        def chunk_copies(i, do):
            sem = sems[i % 3]
            r0 = t0 + i * _CH
            adj = i % 3 < 2
            for k, d in _SC_DELAYS:
                ksl = pl.ds(k * D, D)
                if i == 0:
                    @pl.when(c > 0)
                    def _body(k=k, d=d, ksl=ksl):
                        if d == _CH:
                            do(win(-1), out_hbm.at[b, pl.ds(r0, _CH), ksl], sem)
                        else:
                            do(ring.at[pl.ds(_CH - d, _CH)],
                               out_hbm.at[b, pl.ds(r0, _CH), ksl], sem)

                    @pl.when(c == 0)
                    def _head(k=k, d=d, ksl=ksl):
                        if d == _CH:
                            do(win(0), out_hbm.at[b, pl.ds(0, _CH), ksl], sem)
                        else:
                            do(win(0).at[pl.ds(0, d)],
                               out_hbm.at[b, pl.ds(0, d), ksl], sem)
                            do(win(0).at[pl.ds(0, _CH - d)],
                               out_hbm.at[b, pl.ds(d, _CH - d), ksl], sem)
                else:
                    if d == _CH:
                        do(win(i - 1), out_hbm.at[b, pl.ds(r0, _CH), ksl], sem)
                    elif adj:
                        do(ring.at[pl.ds((i % 3) * _CH + _CH - d, _CH)],
                           out_hbm.at[b, pl.ds(r0, _CH), ksl], sem)
                    else:
                        do(win(i - 1).at[pl.ds(_CH - d, d)],
                           out_hbm.at[b, pl.ds(r0, d), ksl], sem)
                        do(win(i).at[pl.ds(0, _CH - d)],
                           out_hbm.at[b, pl.ds(r0 + d, _CH - d), ksl], sem)

        def issue(src, dst, sem):
            pltpu.async_copy(src, dst, sem)

        def drain(src, dst, sem):
            pltpu.make_async_copy(src, dst, sem).wait()

        @pl.when(c > 0)
        def _halo():
            pltpu.sync_copy(emb_hbm.at[b, pl.ds(t0 - _CH, _CH), :], win(-1))

        pltpu.sync_copy(emb_hbm.at[b, pl.ds(t0, _CH), :], win(0))
        chunk_copies(0, issue)
        for i in range(1, _NCH):
            if i >= 2:
                chunk_copies(i - 2, drain)
            pltpu.sync_copy(
                emb_hbm.at[b, pl.ds(t0 + i * _CH, _CH), :], win(i))
            chunk_copies(i, issue)

        chunk_copies(_NCH - 2, drain)
        chunk_copies(_NCH - 1, drain)

    return run(embeddings)


def _tc_kernel(emb_ref, halo_ref, out_sc_ref, out_ref):
    del out_sc_ref
    i = pl.program_id(1)
    T = _TC_BLOCK
    cur = emb_ref[0]
    halo = halo_ref[0]
    row = lax.broadcasted_iota(jnp.int32, (T, 1), 0)
    for k, d in _TC_DELAYS:
        shifted = jnp.concatenate([halo[8 - d:], cur[:T - d]], axis=0)
        val = jnp.where((i == 0) & (row < d), cur, shifted)
        out_ref[0, :, k * cur.shape[1]:(k + 1) * cur.shape[1]] = val


def kernel(embeddings):
    B, S, D = embeddings.shape
    T = _TC_BLOCK
    out_sc = _sc_part(embeddings)

    return pl.pallas_call(
        _tc_kernel,
        grid=(B, S // T),
        in_specs=[
            pl.BlockSpec((1, T, D), lambda b, i: (b, i, 0)),
            pl.BlockSpec((1, 8, D),
                         lambda b, i: (b, jnp.maximum(i * (T // 8) - 1, 0), 0)),
            pl.BlockSpec(memory_space=pl.ANY),
        ],
        out_specs=pl.BlockSpec((1, T, 3 * D), lambda b, i: (b, i, 0)),
        out_shape=jax.ShapeDtypeStruct((B, S, _K * D), jnp.float32),
        input_output_aliases={2: 0},
    )(embeddings, embeddings, out_sc)
